# baseline (speedup 1.0000x reference)
import math

import jax
import jax.numpy as jnp
import numpy as np
from jax import lax
from jax.experimental import pallas as pl
from jax.experimental.pallas import tpu as pltpu
from jax.experimental.pallas import tpu_sc as plsc

_NC, _NS, _L = 2, 16, 16
_NW = _NC * _NS

_B, _S, _D = 4, 2048, 1024
_NTOK = _B * _S
_CHUNK = 16
_NBLK = 4
_NCHUNK = _NBLK * _B
_NBUF = 4
_SCALE = math.sqrt(_D)


def _pos_encoding(seq_len, d_model):
    position = np.arange(seq_len, dtype=np.float32)[:, None]
    div_term = np.exp(
        np.arange(0, d_model, 2, dtype=np.float32)
        * (-math.log(10000.0) / d_model))
    pe = np.zeros((seq_len, d_model), dtype=np.float32)
    pe[:, 0::2] = np.sin(position * div_term)
    pe[:, 1::2] = np.cos(position * div_term)
    return pe


def _pe_bf16_words():
    import ml_dtypes
    pe = _pos_encoding(_S, _D)
    pe = pe.reshape(_S, _D // 32, 2, 16).transpose(0, 1, 3, 2).reshape(_S, _D)
    bf = pe.astype(ml_dtypes.bfloat16)
    return bf.reshape(-1).view(np.int32)


_PE_BF = _pe_bf16_words()


@jax.jit
def _embed(idx_arr, table, pe_bf):
    mesh = plsc.VectorSubcoreMesh(
        core_axis_name="c", subcore_axis_name="s",
        num_cores=_NC, num_subcores=_NS)

    @pl.kernel(
        out_type=jax.ShapeDtypeStruct((_NTOK, _D), jnp.float32),
        mesh=mesh,
        scratch_types=[
            pltpu.VMEM((_NCHUNK * _CHUNK,), jnp.int32),
            pltpu.VMEM((_NBUF, _CHUNK, _D), jnp.float32),
            pltpu.VMEM((_CHUNK * _D,), jnp.int32),
            [pltpu.SemaphoreType.DMA] * _NBUF,
            [pltpu.SemaphoreType.DMA] * _NBUF,
            [pltpu.SemaphoreType.DMA] * 2,
        ],
    )
    def body(idx_hbm, table_hbm, pe_hbm, out_hbm,
             idx_v, rows, pe_v, gsems, ssems, pesems):
        cid = lax.axis_index("c")
        sid = lax.axis_index("s")
        wid = sid * _NC + cid
        pltpu.sync_copy(idx_hbm.at[wid], idx_v)

        _PEW = _CHUNK * _D // 2

        def load_pe(blk):
            base = pl.multiple_of(
                (_NBLK * _CHUNK * wid + blk * _CHUNK) * (_D // 2), _PEW)
            return pltpu.async_copy(
                pe_hbm.at[pl.ds(base, _PEW)],
                pe_v.at[pl.ds((blk % 2) * _PEW, _PEW)],
                pesems[blk % 2])

        def gather(j):
            return pltpu.async_copy(
                table_hbm.at[idx_v.at[pl.ds(j * _CHUNK, _CHUNK)]],
                rows.at[j % _NBUF], gsems[j % _NBUF])

        def fma(j):
            buf = j % _NBUF
            pb = (j // _B) % 2

            def row_fma(r, carry):
                for g in range(_D // 32):
                    w = pe_v[pl.ds(
                        pb * _PEW + r * (_D // 2) + g * 16, 16)]
                    pa = lax.bitcast_convert_type(
                        lax.shift_left(w, jnp.int32(16)), jnp.float32)
                    pb16 = lax.bitcast_convert_type(
                        lax.bitwise_and(w, jnp.int32(-65536)), jnp.float32)
                    sa = pl.ds(g * 32, 16)
                    sb = pl.ds(g * 32 + 16, 16)
                    rows[buf, r, sa] = rows[buf, r, sa] * _SCALE + pa
                    rows[buf, r, sb] = rows[buf, r, sb] * _SCALE + pb16
                return carry
            lax.fori_loop(0, _CHUNK, row_fma, 0)

        def store(j):
            blk, batch = divmod(j, _B)
            out_base = batch * _S + _NBLK * _CHUNK * wid + blk * _CHUNK
            return pltpu.async_copy(
                rows.at[j % _NBUF], out_hbm.at[pl.ds(out_base, _CHUNK)],
                ssems[j % _NBUF])

        pe_d = [load_pe(0), load_pe(1)]
        g_d = [None] * _NCHUNK
        s_d = [None] * _NCHUNK
        g_d[0] = gather(0)
        g_d[1] = gather(1)
        for k in range(_NCHUNK):
            if k + 2 < _NCHUNK:
                if k - 2 >= 0:
                    s_d[k - 2].wait()
                g_d[k + 2] = gather(k + 2)
            g_d[k].wait()
            if k % _B == 0:
                pe_d[(k // _B) % 2].wait()
            fma(k)
            s_d[k] = store(k)
            if k % _B == _B - 1 and k // _B + 2 < _NBLK:
                pe_d[k // _B % 2] = load_pe(k // _B + 2)
        for j in range(_NCHUNK - _NBUF, _NCHUNK):
            s_d[j].wait()

    return body(idx_arr, table, pe_bf)


def kernel(x, table):
    pe_bf = jnp.asarray(_PE_BF)
    idx_arr = (x.astype(jnp.int32)
               .reshape(_B, _NW, _NBLK, _CHUNK)
               .transpose(1, 2, 0, 3)
               .reshape(_NW, _NCHUNK * _CHUNK))
    out = _embed(idx_arr, table, pe_bf)
    return out.reshape(_B, _S, _D)
